# baseline (speedup 1.0000x reference)
import functools

import jax
import jax.numpy as jnp
from jax import lax
from jax.experimental import pallas as pl
from jax.experimental.pallas import tpu as pltpu
from jax.experimental.pallas import tpu_sc as plsc

N = 10000
D = 128
NP = 10240
NB = NP // 128
NC, NS = 2, 16
NW = NC * NS
B = 128
PAD_ROW = N + 16

_mesh = plsc.VectorSubcoreMesh(core_axis_name="c", subcore_axis_name="s")


def _deg_body(dst_hbm, zeros_hbm, out_hbm, dst_v, hist_v, idx80_v, bounce_v,
              shared_deg):
    c = lax.axis_index("c")
    s = lax.axis_index("s")
    wid = s * NC + c
    epw = dst_hbm.shape[0] // NW
    pltpu.sync_copy(dst_hbm.at[pl.ds(wid * epw, epw)], dst_v)
    pltpu.sync_copy(zeros_hbm.at[pl.ds(0, NB)], hist_v)

    ones16 = jnp.ones((16,), jnp.float32)

    def hbody(i, carry):
        idx16 = dst_v[pl.ds(i * 16, 16)]
        r = lax.shift_right_logical(idx16, 7)
        col = lax.bitwise_and(idx16, 127)
        plsc.addupdate_scatter(hist_v, [r, col], ones16)
        return carry

    lax.fori_loop(0, epw // 16, hbody, 0)

    @pl.when(s == 0)
    def _():
        pltpu.sync_copy(zeros_hbm.at[pl.ds(0, NB)], shared_deg)

    plsc.subcore_barrier()

    def ibody(i, carry):
        idx80_v[pl.ds(i * 16, 16)] = lax.iota(jnp.int32, 16) + i * 16
        return carry

    lax.fori_loop(0, NB // 16, ibody, 0)
    pltpu.sync_copy(hist_v, shared_deg.at[idx80_v], add=True)
    plsc.subcore_barrier()

    @pl.when(s == 0)
    def _():
        pltpu.sync_copy(shared_deg, bounce_v)
        pltpu.sync_copy(bounce_v, out_hbm.at[c])


def _deg_kernel(dst_flat, zeros128):
    epw = dst_flat.shape[0] // NW
    f = functools.partial(
        pl.kernel,
        out_type=jax.ShapeDtypeStruct((NC, NB, 128), jnp.float32),
        mesh=_mesh,
        scratch_types=[
            pltpu.VMEM((epw,), jnp.int32),
            pltpu.VMEM((NB, 128), jnp.float32),
            pltpu.VMEM((NB,), jnp.int32),
            pltpu.VMEM((NB, 128), jnp.float32),
            pltpu.VMEM_SHARED((NB, 128), jnp.float32),
        ],
        compiler_params=pltpu.CompilerParams(needs_layout_passes=False),
    )(_deg_body)
    return f(dst_flat, zeros128)


EB = 128
NBUF = 8
PREF = 3
BLK = 16
DH = D // NC
CPW = 327680 // (NS * EB)
NBLK = CPW // BLK


def _spmm_body(h0_hbm, h1_hbm, src_hbm, dst_hbm, zeros_hbm, out_hbm,
               iring_s, iring_d, bufs, gsem, ssem, issem, idsem, acc):
    c = lax.axis_index("c")
    s = lax.axis_index("s")
    base_row = s * CPW
    rows_per_tile = NP // NS

    def zbody(i, carry):
        pltpu.sync_copy(zeros_hbm, acc.at[pl.ds(s * rows_per_tile + i * 128, 128)])
        return carry

    lax.fori_loop(0, rows_per_tile // 128, zbody, 0)
    plsc.subcore_barrier()

    def pipe(h_hbm):
        pltpu.async_copy(src_hbm.at[pl.ds(base_row, BLK)], iring_s.at[0],
                         issem.at[0])
        pltpu.async_copy(dst_hbm.at[pl.ds(base_row, BLK)], iring_d.at[0],
                         idsem.at[0])
        pltpu.make_async_copy(src_hbm.at[pl.ds(base_row, BLK)], iring_s.at[0],
                              issem.at[0]).wait()
        pltpu.make_async_copy(dst_hbm.at[pl.ds(base_row, BLK)], iring_d.at[0],
                              idsem.at[0]).wait()
        for b in range(PREF):
            pltpu.async_copy(h_hbm.at[iring_s.at[0, b]], bufs.at[b],
                             gsem.at[b])

        def tbody(t, carry):
            for half in range(2):
                g = 2 * t + half
                j0 = g * BLK
                for b in range(BLK):
                    j = j0 + b
                    bb = b % NBUF
                    bf = (b + PREF) % NBUF
                    jf = j + PREF
                    if b == 5:
                        @pl.when(g + 1 < NBLK)
                        def _():
                            row = base_row + (g + 1) * BLK
                            pltpu.async_copy(src_hbm.at[pl.ds(row, BLK)],
                                             iring_s.at[1 - half],
                                             issem.at[1 - half])
                            pltpu.async_copy(dst_hbm.at[pl.ds(row, BLK)],
                                             iring_d.at[1 - half],
                                             idsem.at[1 - half])
                    if b == BLK - PREF:
                        @pl.when(g + 1 < NBLK)
                        def _():
                            pltpu.make_async_copy(
                                src_hbm.at[pl.ds(base_row, BLK)],
                                iring_s.at[1 - half], issem.at[1 - half]).wait()
                            pltpu.make_async_copy(
                                dst_hbm.at[pl.ds(base_row, BLK)],
                                iring_d.at[1 - half], idsem.at[1 - half]).wait()
                    if b < BLK - PREF:
                        fslot, frow = half, b + PREF
                    else:
                        fslot, frow = 1 - half, b + PREF - BLK

                    @pl.when(jf < CPW)
                    def _():
                        @pl.when(jf >= NBUF)
                        def _():
                            pltpu.make_async_copy(
                                bufs.at[bf], acc.at[iring_d.at[fslot, frow]],
                                ssem.at[bf]).wait()
                        pltpu.async_copy(h_hbm.at[iring_s.at[fslot, frow]],
                                         bufs.at[bf], gsem.at[bf])

                    pltpu.make_async_copy(h_hbm.at[iring_s.at[half, b]],
                                          bufs.at[bb], gsem.at[bb]).wait()
                    pltpu.async_copy(bufs.at[bb], acc.at[iring_d.at[half, b]],
                                     ssem.at[bb], add=True)
            return carry

        lax.fori_loop(0, NBLK // 2, tbody, 0)
        for i in range(NBUF):
            pltpu.make_async_copy(bufs.at[i],
                                  acc.at[iring_d.at[1, BLK - NBUF + i]],
                                  ssem.at[i]).wait()

    @pl.when(c == 0)
    def _():
        pipe(h0_hbm)

    @pl.when(c == 1)
    def _():
        pipe(h1_hbm)

    plsc.subcore_barrier()

    def wbody(i, carry):
        base = s * rows_per_tile + i * EB
        pltpu.sync_copy(acc.at[pl.ds(base, EB)], bufs.at[0])
        pltpu.sync_copy(bufs.at[0], out_hbm.at[c, pl.ds(base, EB)])
        return carry

    lax.fori_loop(0, rows_per_tile // EB, wbody, 0)


def _spmm(h0, h1, src3, dst3, zeros64):
    f = functools.partial(
        pl.kernel,
        out_type=jax.ShapeDtypeStruct((NC, NP, DH), jnp.float32),
        mesh=_mesh,
        scratch_types=[
            pltpu.VMEM((2, BLK, EB), jnp.int32),
            pltpu.VMEM((2, BLK, EB), jnp.int32),
            pltpu.VMEM((NBUF, EB, DH), jnp.float32),
            pltpu.SemaphoreType.DMA((NBUF,)),
            pltpu.SemaphoreType.DMA((NBUF,)),
            pltpu.SemaphoreType.DMA((2,)),
            pltpu.SemaphoreType.DMA((2,)),
            pltpu.VMEM_SHARED((NP, DH), jnp.float32),
        ],
        compiler_params=pltpu.CompilerParams(use_tc_tiling_on_sc=False),
    )(_spmm_body)
    return f(h0, h1, src3, dst3, zeros64)


def _rsqrt_body(d_ref, o_ref):
    dtot = d_ref[0] + d_ref[1] + 1.0
    r = lax.rsqrt(dtot)
    flat = (lax.broadcasted_iota(jnp.int32, (NB, 128), 0) * 128
            + lax.broadcasted_iota(jnp.int32, (NB, 128), 1))
    o_ref[...] = jnp.where(flat < N, r, 0.0)


def _first_body(z_ref, w_ref, dv_ref, o0_ref, o1_ref):
    hp = dv_ref[...] * jnp.dot(z_ref[...], w_ref[...],
                               preferred_element_type=jnp.float32)
    o0_ref[...] = hp[:, :DH]
    o1_ref[...] = hp[:, DH:]


def _mid_body(acc_ref, hp0_ref, hp1_ref, dv_ref, b_ref, g_ref, be_ref,
              wn_ref, o0_ref, o1_ref):
    af = jnp.concatenate([acc_ref[0] + hp0_ref[...],
                          acc_ref[1] + hp1_ref[...]], axis=1)
    conv = dv_ref[...] * af + b_ref[...]
    mask = lax.broadcasted_iota(jnp.int32, (NP, 1), 0) < N
    cm = jnp.where(mask, conv, 0.0)
    mean = jnp.sum(cm, axis=0, keepdims=True) * (1.0 / N)
    dlt = conv - mean
    var = jnp.sum(jnp.where(mask, dlt * dlt, 0.0), axis=0, keepdims=True) * (1.0 / N)
    y = dlt * lax.rsqrt(var + 1e-5) * g_ref[...] + be_ref[...]
    x = jnp.where(mask, jnp.maximum(y, 0.0), 0.0)
    hn = dv_ref[...] * jnp.dot(x, wn_ref[...],
                               preferred_element_type=jnp.float32)
    o0_ref[...] = hn[:, :DH]
    o1_ref[...] = hn[:, DH:]


def _final_body(acc_ref, hp0_ref, hp1_ref, dv_ref, b_ref, o_ref):
    af = jnp.concatenate([acc_ref[0] + hp0_ref[...],
                          acc_ref[1] + hp1_ref[...]], axis=1)
    o_ref[...] = dv_ref[...] * af + b_ref[...]


def _tc(body, out_shape, *args):
    return pl.pallas_call(body, out_shape=out_shape)(*args)


def kernel(z, W1, b1, g1, be1, W2, b2, g2, be2, W3, b3, edge_index):
    E = edge_index.shape[1]
    epad = NW * B * 8 * -(-E // (NW * B * 8))
    src = edge_index[0].astype(jnp.int32)
    dst = edge_index[1].astype(jnp.int32)
    fill = N + jnp.arange(epad - E, dtype=jnp.int32) % (NP - N)
    srcp = jnp.concatenate([src, fill])
    dstp = jnp.concatenate([dst, fill])
    src3 = srcp.reshape(epad // EB, EB)
    dst3 = dstp.reshape(epad // EB, EB)
    zeros128 = jnp.zeros((128, 128), jnp.float32)
    zeros64 = jnp.zeros((128, DH), jnp.float32)
    z_pad = jnp.pad(z, ((0, NP - N), (0, 0)))

    deg2 = _deg_kernel(dstp, zeros128)
    dinv80 = _tc(_rsqrt_body, jax.ShapeDtypeStruct((NB, 128), jnp.float32), deg2)
    dv = dinv80.reshape(NP)[:, None]

    f32 = jnp.float32
    half_t = [jax.ShapeDtypeStruct((NP, DH), f32)] * 2
    h1a, h1b = _tc(_first_body, half_t, z_pad, W1, dv)
    a1 = _spmm(h1a, h1b, src3, dst3, zeros64)
    h2a, h2b = _tc(_mid_body, half_t,
                   a1, h1a, h1b, dv, b1[None], g1[None], be1[None], W2)
    a2 = _spmm(h2a, h2b, src3, dst3, zeros64)
    h3a, h3b = _tc(_mid_body, half_t,
                   a2, h2a, h2b, dv, b2[None], g2[None], be2[None], W3)
    a3 = _spmm(h3a, h3b, src3, dst3, zeros64)
    out = _tc(_final_body, jax.ShapeDtypeStruct((NP, D), f32),
              a3, h3a, h3b, dv, b3[None])
    return out[:N]


# speedup vs baseline: 1.1309x; 1.1309x over previous
import functools

import jax
import jax.numpy as jnp
from jax import lax
from jax.experimental import pallas as pl
from jax.experimental.pallas import tpu as pltpu
from jax.experimental.pallas import tpu_sc as plsc

N = 10000
D = 128
NP = 10240
NB = NP // 128
NC, NS = 2, 16
NW = NC * NS
B = 128
PAD_ROW = N + 16

_mesh = plsc.VectorSubcoreMesh(core_axis_name="c", subcore_axis_name="s")


def _deg_body(dst_hbm, zeros_hbm, out_hbm, dst_v, hist_v, idx80_v, bounce_v,
              shared_deg):
    c = lax.axis_index("c")
    s = lax.axis_index("s")
    wid = s * NC + c
    epw = dst_hbm.shape[0] // NW
    pltpu.sync_copy(dst_hbm.at[pl.ds(wid * epw, epw)], dst_v)
    pltpu.sync_copy(zeros_hbm.at[pl.ds(0, NB)], hist_v)

    ones16 = jnp.ones((16,), jnp.float32)

    def hbody(i, carry):
        idx16 = dst_v[pl.ds(i * 16, 16)]
        r = lax.shift_right_logical(idx16, 7)
        col = lax.bitwise_and(idx16, 127)
        plsc.addupdate_scatter(hist_v, [r, col], ones16)
        return carry

    lax.fori_loop(0, epw // 16, hbody, 0)

    @pl.when(s == 0)
    def _():
        pltpu.sync_copy(zeros_hbm.at[pl.ds(0, NB)], shared_deg)

    plsc.subcore_barrier()

    def ibody(i, carry):
        idx80_v[pl.ds(i * 16, 16)] = lax.iota(jnp.int32, 16) + i * 16
        return carry

    lax.fori_loop(0, NB // 16, ibody, 0)
    pltpu.sync_copy(hist_v, shared_deg.at[idx80_v], add=True)
    plsc.subcore_barrier()

    @pl.when(s == 0)
    def _():
        pltpu.sync_copy(shared_deg, bounce_v)
        pltpu.sync_copy(bounce_v, out_hbm.at[c])


def _deg_kernel(dst_flat, zeros128):
    epw = dst_flat.shape[0] // NW
    f = functools.partial(
        pl.kernel,
        out_type=jax.ShapeDtypeStruct((NC, NB, 128), jnp.float32),
        mesh=_mesh,
        scratch_types=[
            pltpu.VMEM((epw,), jnp.int32),
            pltpu.VMEM((NB, 128), jnp.float32),
            pltpu.VMEM((NB,), jnp.int32),
            pltpu.VMEM((NB, 128), jnp.float32),
            pltpu.VMEM_SHARED((NB, 128), jnp.float32),
        ],
        compiler_params=pltpu.CompilerParams(needs_layout_passes=False),
    )(_deg_body)
    return f(dst_flat, zeros128)


EB = 64
NBUF = 4
PREF = 2
BLK = 8
CPW = 327680 // (NW * EB)
NBLK = CPW // BLK


def _spmm_body(h_hbm, src_hbm, dst_hbm, zeros_hbm, out_hbm,
               iring_s, iring_d, bufs, gsem, ssem, issem, idsem, acc):
    c = lax.axis_index("c")
    s = lax.axis_index("s")
    wid = s * NC + c
    base_row = wid * CPW
    rows_per_tile = NP // NS

    def zbody(i, carry):
        pltpu.sync_copy(zeros_hbm, acc.at[pl.ds(s * rows_per_tile + i * 128, 128)])
        return carry

    lax.fori_loop(0, rows_per_tile // 128, zbody, 0)
    plsc.subcore_barrier()

    pltpu.async_copy(src_hbm.at[pl.ds(base_row, BLK)], iring_s.at[0], issem.at[0])
    pltpu.async_copy(dst_hbm.at[pl.ds(base_row, BLK)], iring_d.at[0], idsem.at[0])
    pltpu.make_async_copy(src_hbm.at[pl.ds(base_row, BLK)], iring_s.at[0],
                          issem.at[0]).wait()
    pltpu.make_async_copy(dst_hbm.at[pl.ds(base_row, BLK)], iring_d.at[0],
                          idsem.at[0]).wait()
    for b in range(PREF):
        pltpu.async_copy(h_hbm.at[iring_s.at[0, b]], bufs.at[b], gsem.at[b])

    def tbody(t, carry):
        for half in range(2):
            g = 2 * t + half
            j0 = g * BLK
            for b in range(BLK):
                j = j0 + b
                bb = b % NBUF
                bf = (b + PREF) % NBUF
                jf = j + PREF
                if b == 2:
                    @pl.when(g + 1 < NBLK)
                    def _():
                        row = base_row + (g + 1) * BLK
                        pltpu.async_copy(src_hbm.at[pl.ds(row, BLK)],
                                         iring_s.at[1 - half],
                                         issem.at[1 - half])
                        pltpu.async_copy(dst_hbm.at[pl.ds(row, BLK)],
                                         iring_d.at[1 - half],
                                         idsem.at[1 - half])
                if b == BLK - PREF:
                    @pl.when(g + 1 < NBLK)
                    def _():
                        pltpu.make_async_copy(
                            src_hbm.at[pl.ds(base_row, BLK)],
                            iring_s.at[1 - half], issem.at[1 - half]).wait()
                        pltpu.make_async_copy(
                            dst_hbm.at[pl.ds(base_row, BLK)],
                            iring_d.at[1 - half], idsem.at[1 - half]).wait()
                if b < BLK - PREF:
                    fslot, frow = half, b + PREF
                else:
                    fslot, frow = 1 - half, b + PREF - BLK

                @pl.when(jf < CPW)
                def _():
                    @pl.when(jf >= NBUF)
                    def _():
                        pltpu.make_async_copy(
                            bufs.at[bf], acc.at[iring_d.at[fslot, frow]],
                            ssem.at[bf]).wait()
                    pltpu.async_copy(h_hbm.at[iring_s.at[fslot, frow]],
                                     bufs.at[bf], gsem.at[bf])

                pltpu.make_async_copy(h_hbm.at[iring_s.at[half, b]],
                                      bufs.at[bb], gsem.at[bb]).wait()
                pltpu.async_copy(bufs.at[bb], acc.at[iring_d.at[half, b]],
                                 ssem.at[bb], add=True)
        return carry

    lax.fori_loop(0, NBLK // 2, tbody, 0)
    for i in range(NBUF):
        pltpu.make_async_copy(bufs.at[i], acc.at[iring_d.at[1, BLK - NBUF + i]],
                              ssem.at[i]).wait()
    plsc.subcore_barrier()

    def wbody(i, carry):
        base = s * rows_per_tile + i * EB
        pltpu.sync_copy(acc.at[pl.ds(base, EB)], bufs.at[0])
        pltpu.sync_copy(bufs.at[0], out_hbm.at[c, pl.ds(base, EB)])
        return carry

    lax.fori_loop(0, rows_per_tile // EB, wbody, 0)


def _spmm(h, src3, dst3, zeros128):
    f = functools.partial(
        pl.kernel,
        out_type=jax.ShapeDtypeStruct((NC, NP, 128), jnp.float32),
        mesh=_mesh,
        scratch_types=[
            pltpu.VMEM((2, BLK, EB), jnp.int32),
            pltpu.VMEM((2, BLK, EB), jnp.int32),
            pltpu.VMEM((NBUF, EB, 128), jnp.float32),
            pltpu.SemaphoreType.DMA((NBUF,)),
            pltpu.SemaphoreType.DMA((NBUF,)),
            pltpu.SemaphoreType.DMA((2,)),
            pltpu.SemaphoreType.DMA((2,)),
            pltpu.VMEM_SHARED((NP, 128), jnp.float32),
        ],
    )(_spmm_body)
    return f(h, src3, dst3, zeros128)


def _rsqrt_body(d_ref, o_ref):
    dtot = d_ref[0] + d_ref[1] + 1.0
    r = lax.rsqrt(dtot)
    flat = (lax.broadcasted_iota(jnp.int32, (NB, 128), 0) * 128
            + lax.broadcasted_iota(jnp.int32, (NB, 128), 1))
    o_ref[...] = jnp.where(flat < N, r, 0.0)


def _first_body(z_ref, w_ref, dv_ref, o_ref):
    o_ref[...] = dv_ref[...] * jnp.dot(z_ref[...], w_ref[...],
                                       preferred_element_type=jnp.float32)


def _mid_body(acc_ref, hp_ref, dv_ref, b_ref, g_ref, be_ref, wn_ref, o_ref):
    conv = dv_ref[...] * (acc_ref[0] + acc_ref[1] + hp_ref[...]) + b_ref[...]
    mask = lax.broadcasted_iota(jnp.int32, (NP, 1), 0) < N
    cm = jnp.where(mask, conv, 0.0)
    mean = jnp.sum(cm, axis=0, keepdims=True) * (1.0 / N)
    dlt = conv - mean
    var = jnp.sum(jnp.where(mask, dlt * dlt, 0.0), axis=0, keepdims=True) * (1.0 / N)
    y = dlt * lax.rsqrt(var + 1e-5) * g_ref[...] + be_ref[...]
    x = jnp.where(mask, jnp.maximum(y, 0.0), 0.0)
    o_ref[...] = dv_ref[...] * jnp.dot(x, wn_ref[...],
                                       preferred_element_type=jnp.float32)


def _final_body(acc_ref, hp_ref, dv_ref, b_ref, o_ref):
    o_ref[...] = (dv_ref[...] * (acc_ref[0] + acc_ref[1] + hp_ref[...])
                  + b_ref[...])


def _tc(body, out_shape, *args):
    return pl.pallas_call(body, out_shape=out_shape)(*args)


def kernel(z, W1, b1, g1, be1, W2, b2, g2, be2, W3, b3, edge_index):
    E = edge_index.shape[1]
    epad = NW * B * 8 * -(-E // (NW * B * 8))
    src = edge_index[0].astype(jnp.int32)
    dst = edge_index[1].astype(jnp.int32)
    fill = N + jnp.arange(epad - E, dtype=jnp.int32) % (NP - N)
    srcp = jnp.concatenate([src, fill])
    dstp = jnp.concatenate([dst, fill])
    src3 = srcp.reshape(epad // EB, EB)
    dst3 = dstp.reshape(epad // EB, EB)
    zeros128 = jnp.zeros((128, 128), jnp.float32)
    z_pad = jnp.pad(z, ((0, NP - N), (0, 0)))

    deg2 = _deg_kernel(dstp, zeros128)
    dinv80 = _tc(_rsqrt_body, jax.ShapeDtypeStruct((NB, 128), jnp.float32), deg2)
    dv = dinv80.reshape(NP)[:, None]

    f32 = jnp.float32
    h1 = _tc(_first_body, jax.ShapeDtypeStruct((NP, D), f32), z_pad, W1, dv)
    a1 = _spmm(h1, src3, dst3, zeros128)
    h2 = _tc(_mid_body, jax.ShapeDtypeStruct((NP, D), f32),
             a1, h1, dv, b1[None], g1[None], be1[None], W2)
    a2 = _spmm(h2, src3, dst3, zeros128)
    h3 = _tc(_mid_body, jax.ShapeDtypeStruct((NP, D), f32),
             a2, h2, dv, b2[None], g2[None], be2[None], W3)
    a3 = _spmm(h3, src3, dst3, zeros128)
    out = _tc(_final_body, jax.ShapeDtypeStruct((NP, D), f32),
              a3, h3, dv, b3[None])
    return out[:N]


# speedup vs baseline: 1.1422x; 1.0100x over previous
import functools

import jax
import jax.numpy as jnp
from jax import lax
from jax.experimental import pallas as pl
from jax.experimental.pallas import tpu as pltpu
from jax.experimental.pallas import tpu_sc as plsc

N = 10000
D = 128
NP = 10240
NB = NP // 128
NC, NS = 2, 16
NW = NC * NS
B = 128
PAD_ROW = N + 16

_mesh = plsc.VectorSubcoreMesh(core_axis_name="c", subcore_axis_name="s")


def _deg_body(dst_hbm, zeros_hbm, out_hbm, dst_v, hist_v, idx80_v, bounce_v,
              shared_deg):
    c = lax.axis_index("c")
    s = lax.axis_index("s")
    wid = s * NC + c
    epw = dst_hbm.shape[0] // NW
    pltpu.sync_copy(dst_hbm.at[pl.ds(wid * epw, epw)], dst_v)
    pltpu.sync_copy(zeros_hbm.at[pl.ds(0, NB)], hist_v)

    ones16 = jnp.ones((16,), jnp.float32)

    def hbody(i, carry):
        idx16 = dst_v[pl.ds(i * 16, 16)]
        r = lax.shift_right_logical(idx16, 7)
        col = lax.bitwise_and(idx16, 127)
        plsc.addupdate_scatter(hist_v, [r, col], ones16)
        return carry

    lax.fori_loop(0, epw // 16, hbody, 0)

    @pl.when(s == 0)
    def _():
        pltpu.sync_copy(zeros_hbm.at[pl.ds(0, NB)], shared_deg)

    plsc.subcore_barrier()

    def ibody(i, carry):
        idx80_v[pl.ds(i * 16, 16)] = lax.iota(jnp.int32, 16) + i * 16
        return carry

    lax.fori_loop(0, NB // 16, ibody, 0)
    pltpu.sync_copy(hist_v, shared_deg.at[idx80_v], add=True)
    plsc.subcore_barrier()

    @pl.when(s == 0)
    def _():
        pltpu.sync_copy(shared_deg, bounce_v)
        pltpu.sync_copy(bounce_v, out_hbm.at[c])


def _deg_kernel(dst_flat, zeros128):
    epw = dst_flat.shape[0] // NW
    f = functools.partial(
        pl.kernel,
        out_type=jax.ShapeDtypeStruct((NC, NB, 128), jnp.float32),
        mesh=_mesh,
        scratch_types=[
            pltpu.VMEM((epw,), jnp.int32),
            pltpu.VMEM((NB, 128), jnp.float32),
            pltpu.VMEM((NB,), jnp.int32),
            pltpu.VMEM((NB, 128), jnp.float32),
            pltpu.VMEM_SHARED((NB, 128), jnp.float32),
        ],
        compiler_params=pltpu.CompilerParams(needs_layout_passes=False),
    )(_deg_body)
    return f(dst_flat, zeros128)


EB = 64
NBUF = 4
PREF = 2
BLK = 8
CPW = 327680 // (NW * EB)
NBLK = CPW // BLK


def _spmm_body(h_hbm, src_hbm, dst_hbm, zeros_hbm, out_hbm,
               iring_s, iring_d, bufs, gsem, ssem, issem, idsem, acc):
    c = lax.axis_index("c")
    s = lax.axis_index("s")
    wid = s * NC + c
    base_row = wid * CPW
    rows_per_tile = NP // NS

    def zbody(i, carry):
        pltpu.sync_copy(zeros_hbm, acc.at[pl.ds(s * rows_per_tile + i * 128, 128)])
        return carry

    lax.fori_loop(0, rows_per_tile // 128, zbody, 0)
    plsc.subcore_barrier()

    pltpu.async_copy(src_hbm.at[pl.ds(base_row, BLK)], iring_s.at[0], issem.at[0])
    pltpu.async_copy(dst_hbm.at[pl.ds(base_row, BLK)], iring_d.at[0], idsem.at[0])
    pltpu.make_async_copy(src_hbm.at[pl.ds(base_row, BLK)], iring_s.at[0],
                          issem.at[0]).wait()
    pltpu.make_async_copy(dst_hbm.at[pl.ds(base_row, BLK)], iring_d.at[0],
                          idsem.at[0]).wait()
    for b in range(PREF):
        pltpu.async_copy(h_hbm.at[iring_s.at[0, b]], bufs.at[b], gsem.at[b])

    def tbody(t, carry):
        for half in range(2):
            g = 2 * t + half
            j0 = g * BLK
            for b in range(BLK):
                j = j0 + b
                bb = b % NBUF
                bf = (b + PREF) % NBUF
                jf = j + PREF
                if b == 2:
                    @pl.when(g + 1 < NBLK)
                    def _():
                        row = base_row + (g + 1) * BLK
                        pltpu.async_copy(src_hbm.at[pl.ds(row, BLK)],
                                         iring_s.at[1 - half],
                                         issem.at[1 - half])
                        pltpu.async_copy(dst_hbm.at[pl.ds(row, BLK)],
                                         iring_d.at[1 - half],
                                         idsem.at[1 - half])
                if b == BLK - PREF:
                    @pl.when(g + 1 < NBLK)
                    def _():
                        pltpu.make_async_copy(
                            src_hbm.at[pl.ds(base_row, BLK)],
                            iring_s.at[1 - half], issem.at[1 - half]).wait()
                        pltpu.make_async_copy(
                            dst_hbm.at[pl.ds(base_row, BLK)],
                            iring_d.at[1 - half], idsem.at[1 - half]).wait()
                if b < BLK - PREF:
                    fslot, frow = half, b + PREF
                else:
                    fslot, frow = 1 - half, b + PREF - BLK

                @pl.when(jf < CPW)
                def _():
                    @pl.when(jf >= NBUF)
                    def _():
                        pltpu.make_async_copy(
                            bufs.at[bf], acc.at[iring_d.at[fslot, frow]],
                            ssem.at[bf]).wait()
                    pltpu.async_copy(h_hbm.at[iring_s.at[fslot, frow]],
                                     bufs.at[bf], gsem.at[bf])

                pltpu.make_async_copy(h_hbm.at[iring_s.at[half, b]],
                                      bufs.at[bb], gsem.at[bb]).wait()
                pltpu.async_copy(bufs.at[bb], acc.at[iring_d.at[half, b]],
                                 ssem.at[bb], add=True)
        return carry

    lax.fori_loop(0, NBLK // 2, tbody, 0)
    for i in range(NBUF):
        pltpu.make_async_copy(bufs.at[i], acc.at[iring_d.at[1, BLK - NBUF + i]],
                              ssem.at[i]).wait()
    plsc.subcore_barrier()

    def wbody(i, carry):
        base = s * rows_per_tile + i * EB
        pltpu.sync_copy(acc.at[pl.ds(base, EB)], bufs.at[0])
        pltpu.sync_copy(bufs.at[0], out_hbm.at[c, pl.ds(base, EB)])
        return carry

    lax.fori_loop(0, rows_per_tile // EB, wbody, 0)


def _spmm(h, src3, dst3, zeros128):
    f = functools.partial(
        pl.kernel,
        out_type=jax.ShapeDtypeStruct((NC, NP, 128), jnp.float32),
        mesh=_mesh,
        scratch_types=[
            pltpu.VMEM((2, BLK, EB), jnp.int32),
            pltpu.VMEM((2, BLK, EB), jnp.int32),
            pltpu.VMEM((NBUF, EB, 128), jnp.float32),
            pltpu.SemaphoreType.DMA((NBUF,)),
            pltpu.SemaphoreType.DMA((NBUF,)),
            pltpu.SemaphoreType.DMA((2,)),
            pltpu.SemaphoreType.DMA((2,)),
            pltpu.VMEM_SHARED((NP, 128), jnp.float32),
        ],
    )(_spmm_body)
    return f(h, src3, dst3, zeros128)


def _rsqrt_body(d_ref, o_ref):
    dtot = d_ref[0] + d_ref[1] + 1.0
    r = lax.rsqrt(dtot)
    flat = (lax.broadcasted_iota(jnp.int32, (NB, 128), 0) * 128
            + lax.broadcasted_iota(jnp.int32, (NB, 128), 1))
    o_ref[...] = jnp.where(flat < N, r, 0.0)


def _first_body(z_ref, w_ref, dv_ref, o_ref):
    mm = jnp.dot(z_ref[...], w_ref[...], preferred_element_type=jnp.float32)
    mm_pad = jnp.concatenate([mm, jnp.zeros((NP - N, D), jnp.float32)], axis=0)
    o_ref[...] = dv_ref[...] * mm_pad


def _mid_body(acc_ref, hp_ref, dv_ref, b_ref, g_ref, be_ref, wn_ref, o_ref):
    conv = dv_ref[...] * (acc_ref[0] + acc_ref[1] + hp_ref[...]) + b_ref[...]
    mask = lax.broadcasted_iota(jnp.int32, (NP, 1), 0) < N
    cm = jnp.where(mask, conv, 0.0)
    mean = jnp.sum(cm, axis=0, keepdims=True) * (1.0 / N)
    dlt = conv - mean
    var = jnp.sum(jnp.where(mask, dlt * dlt, 0.0), axis=0, keepdims=True) * (1.0 / N)
    y = dlt * lax.rsqrt(var + 1e-5) * g_ref[...] + be_ref[...]
    x = jnp.where(mask, jnp.maximum(y, 0.0), 0.0)
    o_ref[...] = dv_ref[...] * jnp.dot(x, wn_ref[...],
                                       preferred_element_type=jnp.float32)


def _final_body(acc_ref, hp_ref, dv_ref, b_ref, o_ref):
    full = (dv_ref[...] * (acc_ref[0] + acc_ref[1] + hp_ref[...])
            + b_ref[...])
    o_ref[...] = full[:N]


def _tc(body, out_shape, *args):
    return pl.pallas_call(body, out_shape=out_shape)(*args)


def kernel(z, W1, b1, g1, be1, W2, b2, g2, be2, W3, b3, edge_index):
    E = edge_index.shape[1]
    epad = NW * B * 8 * -(-E // (NW * B * 8))
    src = edge_index[0].astype(jnp.int32)
    dst = edge_index[1].astype(jnp.int32)
    fill = N + jnp.arange(epad - E, dtype=jnp.int32) % (NP - N)
    srcp = jnp.concatenate([src, fill])
    dstp = jnp.concatenate([dst, fill])
    src3 = srcp.reshape(epad // EB, EB)
    dst3 = dstp.reshape(epad // EB, EB)
    zeros128 = jnp.zeros((128, 128), jnp.float32)

    deg2 = _deg_kernel(dstp, zeros128)
    dinv80 = _tc(_rsqrt_body, jax.ShapeDtypeStruct((NB, 128), jnp.float32), deg2)
    dv = dinv80.reshape(NP)[:, None]

    f32 = jnp.float32
    h1 = _tc(_first_body, jax.ShapeDtypeStruct((NP, D), f32), z, W1, dv)
    a1 = _spmm(h1, src3, dst3, zeros128)
    h2 = _tc(_mid_body, jax.ShapeDtypeStruct((NP, D), f32),
             a1, h1, dv, b1[None], g1[None], be1[None], W2)
    a2 = _spmm(h2, src3, dst3, zeros128)
    h3 = _tc(_mid_body, jax.ShapeDtypeStruct((NP, D), f32),
             a2, h2, dv, b2[None], g2[None], be2[None], W3)
    a3 = _spmm(h3, src3, dst3, zeros128)
    out = _tc(_final_body, jax.ShapeDtypeStruct((N, D), f32),
              a3, h3, dv, b3[None])
    return out
